# baseline (speedup 1.0000x reference)
import functools
import math

import jax
import jax.numpy as jnp
from jax import lax
from jax.experimental import pallas as pl
from jax.experimental.pallas import tpu as pltpu
from jax.experimental.pallas import tpu_sc as plsc

NUM_CORES = 2
NUM_SUBCORES = 16
NUM_WORKERS = NUM_CORES * NUM_SUBCORES
LANES = 16

CHUNK = 256
G = 512
NCH = G // CHUNK
NT = G // 2

SCALE_ROWS = 625


def _scale_body(scale, av, cv, api_hbm, cls_hbm, out_hbm,
                b0, b1, b2, zbuf, sin0, sin1, sin2, sot0, sot1, sot2):
    wid = lax.axis_index("s") * NUM_CORES + lax.axis_index("c")
    bufs = (b0, b1, b2)
    sin = (sin0, sin1, sin2)
    sot = (sot0, sot1, sot2)

    def scale_buf(buf, nrows):
        def body(i, carry):
            for j in range(64 // LANES):
                v = buf[i, pl.ds(j * LANES, LANES)]
                buf[i, pl.ds(j * LANES, LANES)] = v * scale
            return carry
        lax.fori_loop(0, nrows, body, 0)

    def zero_row(dst_row):
        zero = jnp.zeros((LANES,), jnp.float32)
        for i in range(64 // LANES):
            zbuf[0, pl.ds(i * LANES, LANES)] = zero
        pltpu.async_copy(zbuf, out_hbm.at[pl.ds(dst_row, 1)], sin0).wait()

    rows_w = av // NUM_WORKERS
    nch = rows_w // SCALE_ROWS
    base = wid * rows_w

    def fire_in(c, b):
        pltpu.async_copy(
            api_hbm.at[pl.ds(base + c * SCALE_ROWS, SCALE_ROWS)],
            bufs[b], sin[b])

    def wait_in(c, b):
        pltpu.make_async_copy(
            api_hbm.at[pl.ds(base + c * SCALE_ROWS, SCALE_ROWS)],
            bufs[b], sin[b]).wait()

    def fire_out(c, b):
        pltpu.async_copy(
            bufs[b], out_hbm.at[pl.ds(base + c * SCALE_ROWS, SCALE_ROWS)],
            sot[b])

    def wait_out(c, b):
        pltpu.make_async_copy(
            bufs[b], out_hbm.at[pl.ds(base + c * SCALE_ROWS, SCALE_ROWS)],
            sot[b]).wait()

    for c in range(min(3, nch)):
        fire_in(c, c)
    for c in range(nch):
        b = c % 3
        wait_in(c, b)
        scale_buf(bufs[b], SCALE_ROWS)
        fire_out(c, b)
        if c + 3 < nch:
            wait_out(c, b)
            fire_in(c + 3, b)
    for c in range(max(0, nch - 3), nch):
        wait_out(c, c % 3)

    cls_w = -(-cv // NUM_WORKERS)
    cbase = wid * cls_w

    def cls_span(nrows):
        pltpu.async_copy(
            cls_hbm.at[pl.ds(cbase, nrows)],
            b0.at[pl.ds(0, nrows), :], sin0).wait()
        scale_buf(b0, nrows)
        pltpu.async_copy(
            b0.at[pl.ds(0, nrows), :],
            out_hbm.at[pl.ds(av + cbase, nrows)], sin0).wait()

    crem = cv - cls_w * (NUM_WORKERS - 1)

    @pl.when(wid < NUM_WORKERS - 1)
    def _():
        cls_span(cls_w)

    @pl.when(wid == NUM_WORKERS - 1)
    def _():
        cls_span(crem)

    @pl.when(wid == 0)
    def _():
        zero_row(0)
        zero_row(av)


def _build_table(api_table, class_table, scale):
    av, d = api_table.shape
    cv = class_table.shape[0]
    mesh = plsc.VectorSubcoreMesh(
        core_axis_name="c", subcore_axis_name="s",
        num_cores=NUM_CORES, num_subcores=NUM_SUBCORES)
    return pl.kernel(
        functools.partial(_scale_body, scale, av, cv),
        out_type=jax.ShapeDtypeStruct((av + cv, d), jnp.float32),
        mesh=mesh,
        scratch_types=[
            pltpu.VMEM((SCALE_ROWS, d), jnp.float32),
            pltpu.VMEM((SCALE_ROWS, d), jnp.float32),
            pltpu.VMEM((SCALE_ROWS, d), jnp.float32),
            pltpu.VMEM((1, d), jnp.float32),
            pltpu.SemaphoreType.DMA,
            pltpu.SemaphoreType.DMA,
            pltpu.SemaphoreType.DMA,
            pltpu.SemaphoreType.DMA,
            pltpu.SemaphoreType.DMA,
            pltpu.SemaphoreType.DMA,
        ],
        compiler_params=pltpu.CompilerParams(use_tc_tiling_on_sc=False, needs_layout_passes=False),
    )(api_table, class_table)


def _gather_body(rows_per_w, av, tbl_hbm, cls_hbm, api_hbm, out_hbm,
                 cls_v, api_v, idx_v, rows_v, si0, si1, sg0, sg1, so0, so1):
    wid = lax.axis_index("s") * NUM_CORES + lax.axis_index("c")
    w0 = wid * rows_per_w
    t0 = wid * (rows_per_w // 2)
    ngroups = rows_per_w // G
    si = (si0, si1)
    sg = (sg0, sg1)
    so = (so0, so1)
    pos0 = lax.iota(jnp.int32, LANES) * 2

    def idx_copy(g, s):
        tb = t0 + g * NT
        pltpu.async_copy(cls_hbm.at[pl.ds(tb, NT)], cls_v.at[s], si[s])
        pltpu.async_copy(api_hbm.at[pl.ds(tb, NT)], api_v.at[s], si[s])

    def interleave(s):
        def body(i, carry):
            c = cls_v[s, pl.ds(i * LANES, LANES)] + av
            a = api_v[s, pl.ds(i * LANES, LANES)]
            pos = pos0 + i * (2 * LANES)
            plsc.store_scatter(idx_v.at[s], [pos], c)
            plsc.store_scatter(idx_v.at[s], [pos + 1], a)
            return carry
        lax.fori_loop(0, NT // LANES, body, 0)

    def fire_group(g, s):
        tb = t0 + g * NT
        pltpu.make_async_copy(
            cls_hbm.at[pl.ds(tb, NT)], cls_v.at[s], si[s]).wait()
        pltpu.make_async_copy(
            api_hbm.at[pl.ds(tb, NT)], api_v.at[s], si[s]).wait()
        interleave(s)
        for j in range(NCH):
            pltpu.async_copy(
                tbl_hbm.at[idx_v.at[s, pl.ds(j * CHUNK, CHUNK)]],
                rows_v.at[s, pl.ds(j * CHUNK, CHUNK), :], sg[s])

    def drain_gathers(s):
        for j in range(NCH):
            pltpu.make_async_copy(
                tbl_hbm.at[idx_v.at[s, pl.ds(j * CHUNK, CHUNK)]],
                rows_v.at[s, pl.ds(j * CHUNK, CHUNK), :], sg[s]).wait()

    def fire_out(g, s):
        pltpu.async_copy(rows_v.at[s], out_hbm.at[pl.ds(w0 + g * G, G)], so[s])

    def wait_out(g, s):
        pltpu.make_async_copy(
            rows_v.at[s], out_hbm.at[pl.ds(w0 + g * G, G)], so[s]).wait()

    for s in range(2):
        idx_copy(s, s)
    fire_group(0, 0)
    idx_copy(2, 0)
    fire_group(1, 1)
    drain_gathers(0)
    fire_out(0, 0)
    idx_copy(3, 1)

    def outer(g2, carry):
        for s in range(2):
            g = 2 * g2 + s
            wait_out(g - 2, s)
            fire_group(g, s)
            drain_gathers(1 - s)
            fire_out(g - 1, 1 - s)
            idx_copy(g + 2, s)
        return carry

    lax.fori_loop(1, ngroups // 2 - 1, outer, 0)

    for g in (ngroups - 2, ngroups - 1):
        s = g % 2
        wait_out(g - 2, s)
        fire_group(g, s)
        drain_gathers(1 - s)
        fire_out(g - 1, 1 - s)
    drain_gathers((ngroups - 1) % 2)
    fire_out(ngroups - 1, (ngroups - 1) % 2)
    for g in (ngroups - 2, ngroups - 1):
        wait_out(g, g % 2)


def _gather_rows(tbl, cls_flat, api_flat, av, n_rows, d):
    rows_per_w = n_rows // NUM_WORKERS
    mesh = plsc.VectorSubcoreMesh(
        core_axis_name="c", subcore_axis_name="s",
        num_cores=NUM_CORES, num_subcores=NUM_SUBCORES)
    return pl.kernel(
        functools.partial(_gather_body, rows_per_w, av),
        out_type=jax.ShapeDtypeStruct((n_rows, d), jnp.float32),
        mesh=mesh,
        scratch_types=[
            pltpu.VMEM((2, NT), jnp.int32),
            pltpu.VMEM((2, NT), jnp.int32),
            pltpu.VMEM((2, G), jnp.int32),
            pltpu.VMEM((2, G, d), jnp.float32),
            pltpu.SemaphoreType.DMA,
            pltpu.SemaphoreType.DMA,
            pltpu.SemaphoreType.DMA,
            pltpu.SemaphoreType.DMA,
            pltpu.SemaphoreType.DMA,
            pltpu.SemaphoreType.DMA,
        ],
        compiler_params=pltpu.CompilerParams(use_tc_tiling_on_sc=False, needs_layout_passes=False),
    )(tbl, cls_flat, api_flat)


def kernel(class_seq, api_seq, class_table, api_table):
    b, s = class_seq.shape
    av, da = api_table.shape
    d_model = class_table.shape[1] + da
    scale = math.sqrt(float(d_model))
    t = b * s

    tbl = _build_table(api_table, class_table, scale)
    cls_flat = class_seq.reshape(t).astype(jnp.int32)
    api_flat = api_seq.reshape(t).astype(jnp.int32)
    out2 = _gather_rows(tbl, cls_flat, api_flat, av, 2 * t, da)
    return out2.reshape(b, s, d_model)


# speedup vs baseline: 1.0034x; 1.0034x over previous
import functools
import math

import jax
import jax.numpy as jnp
from jax import lax
from jax.experimental import pallas as pl
from jax.experimental.pallas import tpu as pltpu
from jax.experimental.pallas import tpu_sc as plsc

NUM_CORES = 2
NUM_SUBCORES = 16
NUM_WORKERS = NUM_CORES * NUM_SUBCORES
LANES = 16

CHUNK = 320
G = 640
NCH = G // CHUNK
NT = G // 2

SCALE_ROWS = 625


def _scale_body(scale, av, cv, api_hbm, cls_hbm, out_hbm,
                b0, b1, b2, zbuf, sin0, sin1, sin2, sot0, sot1, sot2):
    wid = lax.axis_index("s") * NUM_CORES + lax.axis_index("c")
    bufs = (b0, b1, b2)
    sin = (sin0, sin1, sin2)
    sot = (sot0, sot1, sot2)

    def scale_buf(buf, nrows):
        def body(i, carry):
            for j in range(64 // LANES):
                v = buf[i, pl.ds(j * LANES, LANES)]
                buf[i, pl.ds(j * LANES, LANES)] = v * scale
            return carry
        lax.fori_loop(0, nrows, body, 0)

    def zero_row(dst_row):
        zero = jnp.zeros((LANES,), jnp.float32)
        for i in range(64 // LANES):
            zbuf[0, pl.ds(i * LANES, LANES)] = zero
        pltpu.async_copy(zbuf, out_hbm.at[pl.ds(dst_row, 1)], sin0).wait()

    rows_w = av // NUM_WORKERS
    nch = rows_w // SCALE_ROWS
    base = wid * rows_w

    def fire_in(c, b):
        pltpu.async_copy(
            api_hbm.at[pl.ds(base + c * SCALE_ROWS, SCALE_ROWS)],
            bufs[b], sin[b])

    def wait_in(c, b):
        pltpu.make_async_copy(
            api_hbm.at[pl.ds(base + c * SCALE_ROWS, SCALE_ROWS)],
            bufs[b], sin[b]).wait()

    def fire_out(c, b):
        pltpu.async_copy(
            bufs[b], out_hbm.at[pl.ds(base + c * SCALE_ROWS, SCALE_ROWS)],
            sot[b])

    def wait_out(c, b):
        pltpu.make_async_copy(
            bufs[b], out_hbm.at[pl.ds(base + c * SCALE_ROWS, SCALE_ROWS)],
            sot[b]).wait()

    for c in range(min(3, nch)):
        fire_in(c, c)
    for c in range(nch):
        b = c % 3
        wait_in(c, b)
        scale_buf(bufs[b], SCALE_ROWS)
        fire_out(c, b)
        if c + 3 < nch:
            wait_out(c, b)
            fire_in(c + 3, b)
    for c in range(max(0, nch - 3), nch):
        wait_out(c, c % 3)

    cls_w = -(-cv // NUM_WORKERS)
    cbase = wid * cls_w

    def cls_span(nrows):
        pltpu.async_copy(
            cls_hbm.at[pl.ds(cbase, nrows)],
            b0.at[pl.ds(0, nrows), :], sin0).wait()
        scale_buf(b0, nrows)
        pltpu.async_copy(
            b0.at[pl.ds(0, nrows), :],
            out_hbm.at[pl.ds(av + cbase, nrows)], sin0).wait()

    crem = cv - cls_w * (NUM_WORKERS - 1)

    @pl.when(wid < NUM_WORKERS - 1)
    def _():
        cls_span(cls_w)

    @pl.when(wid == NUM_WORKERS - 1)
    def _():
        cls_span(crem)

    @pl.when(wid == 0)
    def _():
        zero_row(0)
        zero_row(av)


def _build_table(api_table, class_table, scale):
    av, d = api_table.shape
    cv = class_table.shape[0]
    mesh = plsc.VectorSubcoreMesh(
        core_axis_name="c", subcore_axis_name="s",
        num_cores=NUM_CORES, num_subcores=NUM_SUBCORES)
    return pl.kernel(
        functools.partial(_scale_body, scale, av, cv),
        out_type=jax.ShapeDtypeStruct((av + cv, d), jnp.float32),
        mesh=mesh,
        scratch_types=[
            pltpu.VMEM((SCALE_ROWS, d), jnp.float32),
            pltpu.VMEM((SCALE_ROWS, d), jnp.float32),
            pltpu.VMEM((SCALE_ROWS, d), jnp.float32),
            pltpu.VMEM((1, d), jnp.float32),
            pltpu.SemaphoreType.DMA,
            pltpu.SemaphoreType.DMA,
            pltpu.SemaphoreType.DMA,
            pltpu.SemaphoreType.DMA,
            pltpu.SemaphoreType.DMA,
            pltpu.SemaphoreType.DMA,
        ],
        compiler_params=pltpu.CompilerParams(use_tc_tiling_on_sc=False, needs_layout_passes=False),
    )(api_table, class_table)


def _gather_body(rows_per_w, av, tbl_hbm, cls_hbm, api_hbm, out_hbm,
                 cls_v, api_v, idx_v, rows_v, si0, si1, sg0, sg1, so0, so1):
    wid = lax.axis_index("s") * NUM_CORES + lax.axis_index("c")
    w0 = wid * rows_per_w
    t0 = wid * (rows_per_w // 2)
    ngroups = rows_per_w // G
    si = (si0, si1)
    sg = (sg0, sg1)
    so = (so0, so1)
    pos0 = lax.iota(jnp.int32, LANES) * 2

    def idx_copy(g, s):
        tb = t0 + g * NT
        pltpu.async_copy(cls_hbm.at[pl.ds(tb, NT)], cls_v.at[s], si[s])
        pltpu.async_copy(api_hbm.at[pl.ds(tb, NT)], api_v.at[s], si[s])

    def interleave(s):
        def body(i, carry):
            c = cls_v[s, pl.ds(i * LANES, LANES)] + av
            a = api_v[s, pl.ds(i * LANES, LANES)]
            pos = pos0 + i * (2 * LANES)
            plsc.store_scatter(idx_v.at[s], [pos], c)
            plsc.store_scatter(idx_v.at[s], [pos + 1], a)
            return carry
        lax.fori_loop(0, NT // LANES, body, 0)

    def fire_group(g, s):
        tb = t0 + g * NT
        pltpu.make_async_copy(
            cls_hbm.at[pl.ds(tb, NT)], cls_v.at[s], si[s]).wait()
        pltpu.make_async_copy(
            api_hbm.at[pl.ds(tb, NT)], api_v.at[s], si[s]).wait()
        interleave(s)
        for j in range(NCH):
            pltpu.async_copy(
                tbl_hbm.at[idx_v.at[s, pl.ds(j * CHUNK, CHUNK)]],
                rows_v.at[s, pl.ds(j * CHUNK, CHUNK), :], sg[s])

    def drain_gathers(s):
        for j in range(NCH):
            pltpu.make_async_copy(
                tbl_hbm.at[idx_v.at[s, pl.ds(j * CHUNK, CHUNK)]],
                rows_v.at[s, pl.ds(j * CHUNK, CHUNK), :], sg[s]).wait()

    def fire_out(g, s):
        pltpu.async_copy(rows_v.at[s], out_hbm.at[pl.ds(w0 + g * G, G)], so[s])

    def wait_out(g, s):
        pltpu.make_async_copy(
            rows_v.at[s], out_hbm.at[pl.ds(w0 + g * G, G)], so[s]).wait()

    for s in range(2):
        idx_copy(s, s)
    fire_group(0, 0)
    idx_copy(2, 0)
    fire_group(1, 1)
    drain_gathers(0)
    fire_out(0, 0)
    idx_copy(3, 1)

    def outer(g2, carry):
        for s in range(2):
            g = 2 * g2 + s
            wait_out(g - 2, s)
            fire_group(g, s)
            drain_gathers(1 - s)
            fire_out(g - 1, 1 - s)
            idx_copy(g + 2, s)
        return carry

    lax.fori_loop(1, ngroups // 2 - 1, outer, 0)

    for g in (ngroups - 2, ngroups - 1):
        s = g % 2
        wait_out(g - 2, s)
        fire_group(g, s)
        drain_gathers(1 - s)
        fire_out(g - 1, 1 - s)
    drain_gathers((ngroups - 1) % 2)
    fire_out(ngroups - 1, (ngroups - 1) % 2)
    for g in (ngroups - 2, ngroups - 1):
        wait_out(g, g % 2)


def _gather_rows(tbl, cls_flat, api_flat, av, n_rows, d):
    rows_per_w = n_rows // NUM_WORKERS
    mesh = plsc.VectorSubcoreMesh(
        core_axis_name="c", subcore_axis_name="s",
        num_cores=NUM_CORES, num_subcores=NUM_SUBCORES)
    return pl.kernel(
        functools.partial(_gather_body, rows_per_w, av),
        out_type=jax.ShapeDtypeStruct((n_rows, d), jnp.float32),
        mesh=mesh,
        scratch_types=[
            pltpu.VMEM((2, NT), jnp.int32),
            pltpu.VMEM((2, NT), jnp.int32),
            pltpu.VMEM((2, G), jnp.int32),
            pltpu.VMEM((2, G, d), jnp.float32),
            pltpu.SemaphoreType.DMA,
            pltpu.SemaphoreType.DMA,
            pltpu.SemaphoreType.DMA,
            pltpu.SemaphoreType.DMA,
            pltpu.SemaphoreType.DMA,
            pltpu.SemaphoreType.DMA,
        ],
        compiler_params=pltpu.CompilerParams(use_tc_tiling_on_sc=False, needs_layout_passes=False),
    )(tbl, cls_flat, api_flat)


def kernel(class_seq, api_seq, class_table, api_table):
    b, s = class_seq.shape
    av, da = api_table.shape
    d_model = class_table.shape[1] + da
    scale = math.sqrt(float(d_model))
    t = b * s

    tbl = _build_table(api_table, class_table, scale)
    cls_flat = class_seq.reshape(t).astype(jnp.int32)
    api_flat = api_seq.reshape(t).astype(jnp.int32)
    out2 = _gather_rows(tbl, cls_flat, api_flat, av, 2 * t, da)
    return out2.reshape(b, s, d_model)
